# baseline (speedup 1.0000x reference)
import functools

import jax
import jax.numpy as jnp
from jax import lax
from jax.experimental import pallas as pl
from jax.experimental.pallas import tpu as pltpu
from jax.experimental.pallas import tpu_sc as plsc

_LAT, _LON = 1801, 3600
_NQ = 1048576
_NW = 32
_QPW = _NQ // _NW
_B = 2048
_NCH = _QPW // _B
_L = 16


def _sc_body(vals_hbm, xq_hbm, yq_hbm, out_hbm,
             xq_v, yq_v, t_v, u_v, idx0_v, idx1_v, gat0_v, gat1_v, out_v,
             sem0, sem1):
    wid = lax.axis_index("s") * 2 + lax.axis_index("c")
    qbase = wid * _QPW
    sems = (sem0, sem1)
    idxs = (idx0_v, idx1_v)
    gats = (gat0_v, gat1_v)

    def phase_a(g, p):
        base = qbase + g * _B
        pltpu.sync_copy(xq_hbm.at[pl.ds(base, _B)], xq_v.at[p])
        pltpu.sync_copy(yq_hbm.at[pl.ds(base, _B)], yq_v.at[p])

        @plsc.parallel_loop(0, _B // _L, unroll=4)
        def vec_a(v):
            s = v * _L
            xq = xq_v[p, pl.ds(s, _L)]
            f00 = ((xq + 90.0) * 35000.0).astype(jnp.int32)
            idxs[p][pl.ds(s, _L)] = f00
            idxs[p][pl.ds(_B + s, _L)] = f00 + 1
            idxs[p][pl.ds(2 * _B + s, _L)] = f00 + _LON
            idxs[p][pl.ds(3 * _B + s, _L)] = f00 + _LON + 1

        pltpu.async_copy(vals_hbm.at[idxs[p]], gats[p], sems[p])

    def phase_b(g, p):
        pltpu.make_async_copy(vals_hbm.at[pl.ds(0, 4 * _B)], gats[p],
                              sems[p]).wait()

        @plsc.parallel_loop(0, _B // _L, unroll=4)
        def vec_b(v):
            s = v * _L
            out_v[p, pl.ds(s, _L)] = gats[p][pl.ds(s, _L)]

        pltpu.sync_copy(out_v.at[p], out_hbm.at[pl.ds(qbase + g * _B, _B)])

    phase_a(0, 0)

    def pair_body(k, carry):
        for p in (0, 1):
            g = 2 * k + p

            @pl.when(g + 1 < _NCH)
            def _():
                phase_a(g + 1, 1 - p)

            phase_b(g, p)
        return carry

    lax.fori_loop(0, _NCH // 2, pair_body, 0)


@jax.jit
def _interp(vals_flat, xq, yq):
    mesh = plsc.VectorSubcoreMesh(core_axis_name="c", subcore_axis_name="s")
    fn = pl.kernel(
        _sc_body,
        out_type=jax.ShapeDtypeStruct((_NQ,), jnp.float32),
        mesh=mesh,
        scratch_types=[
            pltpu.VMEM((2, _B), jnp.float32),
            pltpu.VMEM((2, _B), jnp.float32),
            pltpu.VMEM((2, _B), jnp.float32),
            pltpu.VMEM((2, _B), jnp.float32),
            pltpu.VMEM((4 * _B,), jnp.int32),
            pltpu.VMEM((4 * _B,), jnp.int32),
            pltpu.VMEM((4 * _B,), jnp.float32),
            pltpu.VMEM((4 * _B,), jnp.float32),
            pltpu.VMEM((2, _B), jnp.float32),
            pltpu.SemaphoreType.DMA,
            pltpu.SemaphoreType.DMA,
        ],
    )
    return fn(vals_flat, xq, yq)


def kernel(values, grid_latitude, grid_longitude, query_latitude, query_longitude):
    del grid_latitude, grid_longitude
    return _interp(values.reshape(-1), query_latitude, query_longitude)


# speedup vs baseline: 1.3360x; 1.3360x over previous
import functools

import jax
import jax.numpy as jnp
from jax import lax
from jax.experimental import pallas as pl
from jax.experimental.pallas import tpu as pltpu
from jax.experimental.pallas import tpu_sc as plsc

_LAT, _LON = 1801, 3600
_NQ = 1048576
_NW = 32
_QPW = _NQ // _NW
_B = 2048
_NCH = _QPW // _B
_L = 16
_RB = 256


def _pack_body(v_ref, o_ref):
    v = v_ref[...]
    vr = jnp.concatenate([v[:, 1:], v[:, :1]], axis=1)
    hi = lax.bitcast_convert_type(v.astype(jnp.bfloat16), jnp.uint16)
    lo = lax.bitcast_convert_type(vr.astype(jnp.bfloat16), jnp.uint16)
    o_ref[...] = (hi.astype(jnp.int32) << 16) | lo.astype(jnp.int32)


def _build_pairs(values):
    grid = (_LAT + _RB - 1) // _RB
    out = pl.pallas_call(
        _pack_body,
        grid=(grid,),
        in_specs=[pl.BlockSpec((_RB, _LON), lambda b: (b, 0))],
        out_specs=pl.BlockSpec((_RB, _LON), lambda b: (b, 0)),
        out_shape=jax.ShapeDtypeStruct((_LAT, _LON), jnp.int32),
    )(values)
    return out.reshape(-1)


def _sc_body(pairs_hbm, xq_hbm, yq_hbm, out_hbm,
             xq_v, yq_v, t_v, u_v, idx0_v, idx1_v, gat0_v, gat1_v, out_v,
             sem0, sem1):
    wid = lax.axis_index("s") * 2 + lax.axis_index("c")
    qbase = wid * _QPW
    sems = (sem0, sem1)
    idxs = (idx0_v, idx1_v)
    gats = (gat0_v, gat1_v)

    def phase_a(g, p):
        base = qbase + g * _B
        pltpu.sync_copy(xq_hbm.at[pl.ds(base, _B)], xq_v.at[p])
        pltpu.sync_copy(yq_hbm.at[pl.ds(base, _B)], yq_v.at[p])

        @plsc.parallel_loop(0, _B // _L, unroll=4)
        def vec_a(v):
            s = v * _L
            xq = xq_v[p, pl.ds(s, _L)]
            yq = yq_v[p, pl.ds(s, _L)]
            fi = (xq + 90.0) * 10.0
            ii = jnp.clip(fi.astype(jnp.int32), 0, _LAT - 2)
            t = fi - ii.astype(jnp.float32)
            fy = (yq + 180.0) * 10.0
            jj = jnp.clip(fy.astype(jnp.int32), 0, _LON - 1)
            u = fy - jj.astype(jnp.float32)
            f00 = ii * _LON + jj
            t_v[p, pl.ds(s, _L)] = t
            u_v[p, pl.ds(s, _L)] = u
            idxs[p][pl.ds(s, _L)] = f00
            idxs[p][pl.ds(_B + s, _L)] = f00 + _LON

        pltpu.async_copy(pairs_hbm.at[idxs[p]], gats[p], sems[p])

    def phase_b(g, p):
        pltpu.make_async_copy(pairs_hbm.at[pl.ds(0, 2 * _B)], gats[p],
                              sems[p]).wait()

        @plsc.parallel_loop(0, _B // _L, unroll=4)
        def vec_b(v):
            s = v * _L
            w0 = gats[p][pl.ds(s, _L)]
            w1 = gats[p][pl.ds(_B + s, _L)]
            m = jnp.int32(-65536)
            v00 = lax.bitcast_convert_type(w0 & m, jnp.float32)
            v01 = lax.bitcast_convert_type(w0 << 16, jnp.float32)
            v10 = lax.bitcast_convert_type(w1 & m, jnp.float32)
            v11 = lax.bitcast_convert_type(w1 << 16, jnp.float32)
            t = t_v[p, pl.ds(s, _L)]
            u = u_v[p, pl.ds(s, _L)]
            a = v00 + u * (v01 - v00)
            b = v10 + u * (v11 - v10)
            out_v[p, pl.ds(s, _L)] = a + t * (b - a)

        pltpu.sync_copy(out_v.at[p], out_hbm.at[pl.ds(qbase + g * _B, _B)])

    phase_a(0, 0)

    def pair_body(k, carry):
        for p in (0, 1):
            g = 2 * k + p

            @pl.when(g + 1 < _NCH)
            def _():
                phase_a(g + 1, 1 - p)

            phase_b(g, p)
        return carry

    lax.fori_loop(0, _NCH // 2, pair_body, 0)


@jax.jit
def _interp(values, xq, yq):
    pairs = _build_pairs(values)
    mesh = plsc.VectorSubcoreMesh(core_axis_name="c", subcore_axis_name="s")
    fn = pl.kernel(
        _sc_body,
        out_type=jax.ShapeDtypeStruct((_NQ,), jnp.float32),
        mesh=mesh,
        scratch_types=[
            pltpu.VMEM((2, _B), jnp.float32),
            pltpu.VMEM((2, _B), jnp.float32),
            pltpu.VMEM((2, _B), jnp.float32),
            pltpu.VMEM((2, _B), jnp.float32),
            pltpu.VMEM((2 * _B,), jnp.int32),
            pltpu.VMEM((2 * _B,), jnp.int32),
            pltpu.VMEM((2 * _B,), jnp.int32),
            pltpu.VMEM((2 * _B,), jnp.int32),
            pltpu.VMEM((2, _B), jnp.float32),
            pltpu.SemaphoreType.DMA,
            pltpu.SemaphoreType.DMA,
        ],
    )
    return fn(pairs, xq, yq)


def kernel(values, grid_latitude, grid_longitude, query_latitude, query_longitude):
    del grid_latitude, grid_longitude
    return _interp(values, query_latitude, query_longitude)


# speedup vs baseline: 1.3553x; 1.0144x over previous
import functools

import jax
import jax.numpy as jnp
from jax import lax
from jax.experimental import pallas as pl
from jax.experimental.pallas import tpu as pltpu
from jax.experimental.pallas import tpu_sc as plsc

_LAT, _LON = 1801, 3600
_NQ = 1048576
_NW = 32
_QPW = _NQ // _NW
_B = 4096
_NCH = _QPW // _B
_L = 16
_RB = 256


def _pack_body(v_ref, o_ref):
    v = v_ref[...]
    vr = jnp.concatenate([v[:, 1:], v[:, :1]], axis=1)
    hi = lax.bitcast_convert_type(v.astype(jnp.bfloat16), jnp.uint16)
    lo = lax.bitcast_convert_type(vr.astype(jnp.bfloat16), jnp.uint16)
    o_ref[...] = (hi.astype(jnp.int32) << 16) | lo.astype(jnp.int32)


def _build_pairs(values):
    grid = (_LAT + _RB - 1) // _RB
    out = pl.pallas_call(
        _pack_body,
        grid=(grid,),
        in_specs=[pl.BlockSpec((_RB, _LON), lambda b: (b, 0))],
        out_specs=pl.BlockSpec((_RB, _LON), lambda b: (b, 0)),
        out_shape=jax.ShapeDtypeStruct((_LAT, _LON), jnp.int32),
    )(values)
    return out.reshape(-1)


def _sc_body(pairs_hbm, xq_hbm, yq_hbm, out_hbm,
             xq_v, yq_v, t_v, u_v, idx0_v, idx1_v, gat0_v, gat1_v, out_v,
             sem0, sem1):
    wid = lax.axis_index("s") * 2 + lax.axis_index("c")
    qbase = wid * _QPW
    sems = (sem0, sem1)
    idxs = (idx0_v, idx1_v)
    gats = (gat0_v, gat1_v)

    def phase_a(g, p):
        base = qbase + g * _B
        pltpu.sync_copy(xq_hbm.at[pl.ds(base, _B)], xq_v.at[p])
        pltpu.sync_copy(yq_hbm.at[pl.ds(base, _B)], yq_v.at[p])

        @plsc.parallel_loop(0, _B // _L, unroll=8)
        def vec_a(v):
            s = v * _L
            xq = xq_v[p, pl.ds(s, _L)]
            yq = yq_v[p, pl.ds(s, _L)]
            fi = (xq + 90.0) * 10.0
            ii = jnp.clip(fi.astype(jnp.int32), 0, _LAT - 2)
            t = fi - ii.astype(jnp.float32)
            fy = (yq + 180.0) * 10.0
            jj = jnp.clip(fy.astype(jnp.int32), 0, _LON - 1)
            u = fy - jj.astype(jnp.float32)
            f00 = ii * _LON + jj
            t_v[p, pl.ds(s, _L)] = t
            u_v[p, pl.ds(s, _L)] = u
            idxs[p][pl.ds(s, _L)] = f00
            idxs[p][pl.ds(_B + s, _L)] = f00 + _LON

        pltpu.async_copy(pairs_hbm.at[idxs[p]], gats[p], sems[p])

    def phase_b(g, p):
        pltpu.make_async_copy(pairs_hbm.at[pl.ds(0, 2 * _B)], gats[p],
                              sems[p]).wait()

        @plsc.parallel_loop(0, _B // _L, unroll=8)
        def vec_b(v):
            s = v * _L
            w0 = gats[p][pl.ds(s, _L)]
            w1 = gats[p][pl.ds(_B + s, _L)]
            m = jnp.int32(-65536)
            v00 = lax.bitcast_convert_type(w0 & m, jnp.float32)
            v01 = lax.bitcast_convert_type(w0 << 16, jnp.float32)
            v10 = lax.bitcast_convert_type(w1 & m, jnp.float32)
            v11 = lax.bitcast_convert_type(w1 << 16, jnp.float32)
            t = t_v[p, pl.ds(s, _L)]
            u = u_v[p, pl.ds(s, _L)]
            a = v00 + u * (v01 - v00)
            b = v10 + u * (v11 - v10)
            out_v[p, pl.ds(s, _L)] = a + t * (b - a)

        pltpu.sync_copy(out_v.at[p], out_hbm.at[pl.ds(qbase + g * _B, _B)])

    phase_a(0, 0)

    def pair_body(k, carry):
        for p in (0, 1):
            g = 2 * k + p

            @pl.when(g + 1 < _NCH)
            def _():
                phase_a(g + 1, 1 - p)

            phase_b(g, p)
        return carry

    lax.fori_loop(0, _NCH // 2, pair_body, 0)


@jax.jit
def _interp(values, xq, yq):
    pairs = _build_pairs(values)
    mesh = plsc.VectorSubcoreMesh(core_axis_name="c", subcore_axis_name="s")
    fn = pl.kernel(
        _sc_body,
        out_type=jax.ShapeDtypeStruct((_NQ,), jnp.float32),
        mesh=mesh,
        scratch_types=[
            pltpu.VMEM((2, _B), jnp.float32),
            pltpu.VMEM((2, _B), jnp.float32),
            pltpu.VMEM((2, _B), jnp.float32),
            pltpu.VMEM((2, _B), jnp.float32),
            pltpu.VMEM((2 * _B,), jnp.int32),
            pltpu.VMEM((2 * _B,), jnp.int32),
            pltpu.VMEM((2 * _B,), jnp.int32),
            pltpu.VMEM((2 * _B,), jnp.int32),
            pltpu.VMEM((2, _B), jnp.float32),
            pltpu.SemaphoreType.DMA,
            pltpu.SemaphoreType.DMA,
        ],
    )
    return fn(pairs, xq, yq)


def kernel(values, grid_latitude, grid_longitude, query_latitude, query_longitude):
    del grid_latitude, grid_longitude
    return _interp(values, query_latitude, query_longitude)
